# Initial kernel scaffold; baseline (speedup 1.0000x reference)
import jax
import jax.numpy as jnp
from jax.experimental import pallas as pl


def kernel(flat_node2pin_start, flat_node2pin, sorted_pin_map):
    raise NotImplementedError("write your pallas kernel here")


# speedup vs baseline: 534.0527x; 534.0527x over previous
import functools

import jax
import jax.numpy as jnp
from jax import lax
from jax.experimental import pallas as pl
from jax.experimental.pallas import tpu as pltpu
from jax.experimental.pallas import tpu_sc as plsc

NN = 100000
NP = 1600000
NW = 32
NPW = 3136
NGRP = NPW // 16
CH = 16384
CROWS = CH // 128
SPAD = 3168
INT_MAX = 2**31 - 1


@functools.partial(
    pl.kernel,
    mesh=plsc.VectorSubcoreMesh(core_axis_name="c", subcore_axis_name="s"),
    compiler_params=pltpu.CompilerParams(needs_layout_passes=False),
    out_type=jax.ShapeDtypeStruct((NW * NPW,), jnp.int32),
    scratch_types=[
        pltpu.VMEM((SPAD,), jnp.int32),
        pltpu.VMEM((CH,), jnp.int32),
        pltpu.VMEM((CH,), jnp.int32),
        pltpu.VMEM((NPW,), jnp.int32),
        pltpu.SemaphoreType.DMA,
    ],
)
def _segmin_kernel(start_hbm, flat_hbm, spm_hbm, out_hbm,
                   sbuf, fbuf, vbuf, obuf, dsem):
    cid = lax.axis_index("c")
    sid = lax.axis_index("s")
    w = sid * 2 + cid
    nbase = w * NPW
    pltpu.sync_copy(start_hbm.at[pl.ds(nbase, SPAD)], sbuf)

    lane = lax.iota(jnp.int32, 16)
    p0 = sbuf[pl.ds(0, 16)][0]
    s0 = plsc.load_gather(sbuf, [lane])
    e0 = plsc.load_gather(sbuf, [lane + 1])
    c0_init = (p0 // 8) * 8

    def outer_cond(st):
        gi, c0, s, e, ptr, accv, accp = st
        return gi < NGRP

    def outer_body(st):
        gi, c0, s, e, ptr, accv, accp = st
        c1 = c0 + CH
        pltpu.sync_copy(flat_hbm.at[pl.ds(pl.multiple_of(c0, 8), CH)], fbuf)

        def fire(r, x):
            pltpu.make_async_copy(
                spm_hbm.at[fbuf.at[pl.ds(r * 128, 128)]],
                vbuf.at[pl.ds(r * 128, 128)], dsem).start()
            return x

        lax.fori_loop(0, CROWS, fire, 0)

        def drain(r, x):
            pltpu.make_async_copy(
                spm_hbm.at[fbuf.at[pl.ds(r * 128, 128)]],
                vbuf.at[pl.ds(r * 128, 128)], dsem).wait()
            return x

        lax.fori_loop(0, CROWS, drain, 0)

        def in_cond(st2):
            go, gi2, s2, e2, ptr2, av2, ap2 = st2
            return go & (gi2 < NGRP)

        def in_body(st2):
            go, gi2, s2, e2, ptr2, av2, ap2 = st2
            active = (ptr2 < e2) & (ptr2 < c1)
            idx = jnp.clip(ptr2 - c0, 0, CH - 1)
            v = plsc.load_gather(vbuf, [idx], mask=active)
            p = plsc.load_gather(fbuf, [idx], mask=active)
            upd = active & (v < av2)
            av3 = jnp.where(upd, v, av2)
            ap3 = jnp.where(upd, p, ap2)
            ptr3 = ptr2 + jnp.where(active, 1, 0).astype(jnp.int32)
            done = ~jnp.any(ptr3 < e2)
            blocked = (~jnp.any((ptr3 < e2) & (ptr3 < c1))) & (~done)

            def emit_adv(args):
                gi_c, s_c, e_c, av_c, ap_c = args
                res = jnp.where(e_c > s_c, ap_c, jnp.int32(0))
                obuf[pl.ds(gi_c * 16, 16)] = res
                gi_n = gi_c + 1
                base = gi_n * 16
                s_n = plsc.load_gather(sbuf, [base + lane])
                e_n = plsc.load_gather(sbuf, [base + lane + 1])
                return (gi_n, s_n, e_n, s_n,
                        jnp.full((16,), INT_MAX, jnp.int32),
                        jnp.zeros((16,), jnp.int32))

            def stay(args):
                gi_c, s_c, e_c, av_c, ap_c = args
                return (gi_c, s_c, e_c, ptr3, av_c, ap_c)

            gi3, s3, e3, ptr4, av4, ap4 = lax.cond(
                done, emit_adv, stay, (gi2, s2, e2, av3, ap3))
            return (~blocked, gi3, s3, e3, ptr4, av4, ap4)

        go, gi2, s2, e2, ptr2, av2, ap2 = lax.while_loop(
            in_cond, in_body,
            (jnp.bool_(True), gi, s, e, ptr, accv, accp))
        return (gi2, c1, s2, e2, ptr2, av2, ap2)

    init = (jnp.int32(0), c0_init, s0, e0, s0,
            jnp.full((16,), INT_MAX, jnp.int32),
            jnp.zeros((16,), jnp.int32))
    lax.while_loop(outer_cond, outer_body, init)

    pltpu.sync_copy(obuf, out_hbm.at[pl.ds(nbase, NPW)])


def kernel(flat_node2pin_start, flat_node2pin, sorted_pin_map):
    start_pad = jnp.pad(flat_node2pin_start,
                        (0, NW * NPW + SPAD - (NN + 1)),
                        constant_values=NP)
    flat_pad = jnp.pad(flat_node2pin, (0, CH + 8))
    out = _segmin_kernel(start_pad, flat_pad, sorted_pin_map)
    return out[:NN]


# speedup vs baseline: 1039.1065x; 1.9457x over previous
import functools

import jax
import jax.numpy as jnp
from jax import lax
from jax.experimental import pallas as pl
from jax.experimental.pallas import tpu as pltpu
from jax.experimental.pallas import tpu_sc as plsc

NN = 100000
NP = 1600000
NW = 32
NPW = 3136
NGRP = NPW // 16
KU = 4
CH = 16384
CROWS = CH // 128
SPAD = 3168
INT_MAX = 2**31 - 1


@functools.partial(
    pl.kernel,
    mesh=plsc.VectorSubcoreMesh(core_axis_name="c", subcore_axis_name="s"),
    compiler_params=pltpu.CompilerParams(needs_layout_passes=False),
    out_type=jax.ShapeDtypeStruct((NW * NPW,), jnp.int32),
    scratch_types=[
        pltpu.VMEM((SPAD,), jnp.int32),
        pltpu.VMEM((CH,), jnp.int32),
        pltpu.VMEM((CH,), jnp.int32),
        pltpu.VMEM((NPW,), jnp.int32),
        pltpu.SemaphoreType.DMA,
    ],
)
def _segmin_kernel(start_hbm, flat_hbm, spm_hbm, out_hbm,
                   sbuf, fbuf, vbuf, obuf, dsem):
    cid = lax.axis_index("c")
    sid = lax.axis_index("s")
    w = sid * 2 + cid
    nbase = w * NPW
    pltpu.sync_copy(start_hbm.at[pl.ds(nbase, SPAD)], sbuf)

    lane = lax.iota(jnp.int32, 16)
    p0 = sbuf[pl.ds(0, 16)][0]
    s0 = plsc.load_gather(sbuf, [lane])
    e0 = plsc.load_gather(sbuf, [lane + 1])
    c0_init = (p0 // 8) * 8

    def outer_cond(st):
        gi, c0, s, e, ptr, accv, accp = st
        return gi < NGRP

    def outer_body(st):
        gi, c0, s, e, ptr, accv, accp = st
        c1 = c0 + CH
        pltpu.sync_copy(flat_hbm.at[pl.ds(pl.multiple_of(c0, 8), CH)], fbuf)

        def fire(r, x):
            pltpu.make_async_copy(
                spm_hbm.at[fbuf.at[pl.ds(r * 128, 128)]],
                vbuf.at[pl.ds(r * 128, 128)], dsem).start()
            return x

        lax.fori_loop(0, CROWS, fire, 0)

        def drain(r, x):
            pltpu.make_async_copy(
                spm_hbm.at[fbuf.at[pl.ds(r * 128, 128)]],
                vbuf.at[pl.ds(r * 128, 128)], dsem).wait()
            return x

        lax.fori_loop(0, CROWS, drain, 0)

        def in_cond(st2):
            go, gi2, s2, e2, ptr2, av2, ap2 = st2
            return go & (gi2 < NGRP)

        def in_body(st2):
            go, gi2, s2, e2, ptr2, av2, ap2 = st2
            ptr3, av3, ap3 = ptr2, av2, ap2
            for _ in range(KU):
                active = (ptr3 < e2) & (ptr3 < c1)
                idx = jnp.clip(ptr3 - c0, 0, CH - 1)
                v = plsc.load_gather(vbuf, [idx], mask=active)
                p = plsc.load_gather(fbuf, [idx], mask=active)
                upd = active & (v < av3)
                av3 = jnp.where(upd, v, av3)
                ap3 = jnp.where(upd, p, ap3)
                ptr3 = ptr3 + jnp.where(active, 1, 0).astype(jnp.int32)
            done = ~jnp.any(ptr3 < e2)
            blocked = (~jnp.any((ptr3 < e2) & (ptr3 < c1))) & (~done)

            def emit_adv(args):
                gi_c, s_c, e_c, av_c, ap_c = args
                res = jnp.where(e_c > s_c, ap_c, jnp.int32(0))
                obuf[pl.ds(gi_c * 16, 16)] = res
                gi_n = gi_c + 1
                base = gi_n * 16
                s_n = plsc.load_gather(sbuf, [base + lane])
                e_n = plsc.load_gather(sbuf, [base + lane + 1])
                return (gi_n, s_n, e_n, s_n,
                        jnp.full((16,), INT_MAX, jnp.int32),
                        jnp.zeros((16,), jnp.int32))

            def stay(args):
                gi_c, s_c, e_c, av_c, ap_c = args
                return (gi_c, s_c, e_c, ptr3, av_c, ap_c)

            gi3, s3, e3, ptr4, av4, ap4 = lax.cond(
                done, emit_adv, stay, (gi2, s2, e2, av3, ap3))
            return (~blocked, gi3, s3, e3, ptr4, av4, ap4)

        go, gi2, s2, e2, ptr2, av2, ap2 = lax.while_loop(
            in_cond, in_body,
            (jnp.bool_(True), gi, s, e, ptr, accv, accp))
        return (gi2, c1, s2, e2, ptr2, av2, ap2)

    init = (jnp.int32(0), c0_init, s0, e0, s0,
            jnp.full((16,), INT_MAX, jnp.int32),
            jnp.zeros((16,), jnp.int32))
    lax.while_loop(outer_cond, outer_body, init)

    pltpu.sync_copy(obuf, out_hbm.at[pl.ds(nbase, NPW)])


def kernel(flat_node2pin_start, flat_node2pin, sorted_pin_map):
    start_pad = jnp.pad(flat_node2pin_start,
                        (0, NW * NPW + SPAD - (NN + 1)),
                        constant_values=NP)
    flat_pad = jnp.pad(flat_node2pin, (0, CH + 8))
    out = _segmin_kernel(start_pad, flat_pad, sorted_pin_map)
    return out[:NN]


# speedup vs baseline: 1211.3537x; 1.1658x over previous
import functools

import jax
import jax.numpy as jnp
from jax import lax
from jax.experimental import pallas as pl
from jax.experimental.pallas import tpu as pltpu
from jax.experimental.pallas import tpu_sc as plsc

NN = 100000
NP = 1600000
NW = 32
NPW = 3136
NGRP = NPW // 16
KU = 8
CH = 16384
CROWS = CH // 128
SPAD = 3168
INT_MAX = 2**31 - 1


@functools.partial(
    pl.kernel,
    mesh=plsc.VectorSubcoreMesh(core_axis_name="c", subcore_axis_name="s"),
    compiler_params=pltpu.CompilerParams(needs_layout_passes=False),
    out_type=jax.ShapeDtypeStruct((NW * NPW,), jnp.int32),
    scratch_types=[
        pltpu.VMEM((SPAD,), jnp.int32),
        pltpu.VMEM((CH,), jnp.int32),
        pltpu.VMEM((CH,), jnp.int32),
        pltpu.VMEM((NPW,), jnp.int32),
        pltpu.SemaphoreType.DMA,
    ],
)
def _segmin_kernel(start_hbm, flat_hbm, spm_hbm, out_hbm,
                   sbuf, fbuf, vbuf, obuf, dsem):
    cid = lax.axis_index("c")
    sid = lax.axis_index("s")
    w = sid * 2 + cid
    nbase = w * NPW
    pltpu.sync_copy(start_hbm.at[pl.ds(nbase, SPAD)], sbuf)

    lane = lax.iota(jnp.int32, 16)
    p0 = sbuf[pl.ds(0, 16)][0]
    s0 = plsc.load_gather(sbuf, [lane])
    e0 = plsc.load_gather(sbuf, [lane + 1])
    c0_init = (p0 // 8) * 8

    def outer_cond(st):
        gi, c0, s, e, ptr, accv, accp = st
        return gi < NGRP

    def outer_body(st):
        gi, c0, s, e, ptr, accv, accp = st
        c1 = c0 + CH
        pltpu.sync_copy(flat_hbm.at[pl.ds(pl.multiple_of(c0, 8), CH)], fbuf)

        def fire(r, x):
            pltpu.make_async_copy(
                spm_hbm.at[fbuf.at[pl.ds(r * 128, 128)]],
                vbuf.at[pl.ds(r * 128, 128)], dsem).start()
            return x

        lax.fori_loop(0, CROWS, fire, 0)

        def drain(r, x):
            pltpu.make_async_copy(
                spm_hbm.at[fbuf.at[pl.ds(r * 128, 128)]],
                vbuf.at[pl.ds(r * 128, 128)], dsem).wait()
            return x

        lax.fori_loop(0, CROWS, drain, 0)

        def in_cond(st2):
            go, gi2, s2, e2, ptr2, av2, ap2 = st2
            return go & (gi2 < NGRP)

        def in_body(st2):
            go, gi2, s2, e2, ptr2, av2, ap2 = st2
            ptr3, av3, ap3 = ptr2, av2, ap2
            for _ in range(KU):
                active = (ptr3 < e2) & (ptr3 < c1)
                idx = jnp.clip(ptr3 - c0, 0, CH - 1)
                v = plsc.load_gather(vbuf, [idx], mask=active)
                p = plsc.load_gather(fbuf, [idx], mask=active)
                upd = active & (v < av3)
                av3 = jnp.where(upd, v, av3)
                ap3 = jnp.where(upd, p, ap3)
                ptr3 = ptr3 + jnp.where(active, 1, 0).astype(jnp.int32)
            done = ~jnp.any(ptr3 < e2)
            blocked = (~jnp.any((ptr3 < e2) & (ptr3 < c1))) & (~done)

            def emit_adv(args):
                gi_c, s_c, e_c, av_c, ap_c = args
                res = jnp.where(e_c > s_c, ap_c, jnp.int32(0))
                obuf[pl.ds(gi_c * 16, 16)] = res
                gi_n = gi_c + 1
                base = gi_n * 16
                s_n = plsc.load_gather(sbuf, [base + lane])
                e_n = plsc.load_gather(sbuf, [base + lane + 1])
                return (gi_n, s_n, e_n, s_n,
                        jnp.full((16,), INT_MAX, jnp.int32),
                        jnp.zeros((16,), jnp.int32))

            def stay(args):
                gi_c, s_c, e_c, av_c, ap_c = args
                return (gi_c, s_c, e_c, ptr3, av_c, ap_c)

            gi3, s3, e3, ptr4, av4, ap4 = lax.cond(
                done, emit_adv, stay, (gi2, s2, e2, av3, ap3))
            return (~blocked, gi3, s3, e3, ptr4, av4, ap4)

        go, gi2, s2, e2, ptr2, av2, ap2 = lax.while_loop(
            in_cond, in_body,
            (jnp.bool_(True), gi, s, e, ptr, accv, accp))
        return (gi2, c1, s2, e2, ptr2, av2, ap2)

    init = (jnp.int32(0), c0_init, s0, e0, s0,
            jnp.full((16,), INT_MAX, jnp.int32),
            jnp.zeros((16,), jnp.int32))
    lax.while_loop(outer_cond, outer_body, init)

    pltpu.sync_copy(obuf, out_hbm.at[pl.ds(nbase, NPW)])


def kernel(flat_node2pin_start, flat_node2pin, sorted_pin_map):
    start_pad = jnp.pad(flat_node2pin_start,
                        (0, NW * NPW + SPAD - (NN + 1)),
                        constant_values=NP)
    flat_pad = jnp.pad(flat_node2pin, (0, CH + 8))
    out = _segmin_kernel(start_pad, flat_pad, sorted_pin_map)
    return out[:NN]


# speedup vs baseline: 1277.5848x; 1.0547x over previous
import functools

import jax
import jax.numpy as jnp
from jax import lax
from jax.experimental import pallas as pl
from jax.experimental.pallas import tpu as pltpu
from jax.experimental.pallas import tpu_sc as plsc

NN = 100000
NP = 1600000
NW = 32
NPW = 3136
NGRP = NPW // 16
KU = 16
CH = 16384
CROWS = CH // 128
SPAD = 3168
INT_MAX = 2**31 - 1


@functools.partial(
    pl.kernel,
    mesh=plsc.VectorSubcoreMesh(core_axis_name="c", subcore_axis_name="s"),
    compiler_params=pltpu.CompilerParams(needs_layout_passes=False),
    out_type=jax.ShapeDtypeStruct((NW * NPW,), jnp.int32),
    scratch_types=[
        pltpu.VMEM((SPAD,), jnp.int32),
        pltpu.VMEM((CH,), jnp.int32),
        pltpu.VMEM((CH,), jnp.int32),
        pltpu.VMEM((NPW,), jnp.int32),
        pltpu.SemaphoreType.DMA,
    ],
)
def _segmin_kernel(start_hbm, flat_hbm, spm_hbm, out_hbm,
                   sbuf, fbuf, vbuf, obuf, dsem):
    cid = lax.axis_index("c")
    sid = lax.axis_index("s")
    w = sid * 2 + cid
    nbase = w * NPW
    pltpu.sync_copy(start_hbm.at[pl.ds(nbase, SPAD)], sbuf)

    lane = lax.iota(jnp.int32, 16)
    p0 = sbuf[pl.ds(0, 16)][0]
    s0 = plsc.load_gather(sbuf, [lane])
    e0 = plsc.load_gather(sbuf, [lane + 1])
    c0_init = (p0 // 8) * 8

    def outer_cond(st):
        gi, c0, s, e, ptr, accv, accp = st
        return gi < NGRP

    def outer_body(st):
        gi, c0, s, e, ptr, accv, accp = st
        c1 = c0 + CH
        pltpu.sync_copy(flat_hbm.at[pl.ds(pl.multiple_of(c0, 8), CH)], fbuf)

        def fire(r, x):
            pltpu.make_async_copy(
                spm_hbm.at[fbuf.at[pl.ds(r * 128, 128)]],
                vbuf.at[pl.ds(r * 128, 128)], dsem).start()
            return x

        lax.fori_loop(0, CROWS, fire, 0)

        def drain(r, x):
            pltpu.make_async_copy(
                spm_hbm.at[fbuf.at[pl.ds(r * 128, 128)]],
                vbuf.at[pl.ds(r * 128, 128)], dsem).wait()
            return x

        lax.fori_loop(0, CROWS, drain, 0)

        def in_cond(st2):
            go, gi2, s2, e2, ptr2, av2, ap2 = st2
            return go & (gi2 < NGRP)

        def in_body(st2):
            go, gi2, s2, e2, ptr2, av2, ap2 = st2
            ptr3, av3, ap3 = ptr2, av2, ap2
            for _ in range(KU):
                active = (ptr3 < e2) & (ptr3 < c1)
                idx = jnp.clip(ptr3 - c0, 0, CH - 1)
                v = plsc.load_gather(vbuf, [idx], mask=active)
                p = plsc.load_gather(fbuf, [idx], mask=active)
                upd = active & (v < av3)
                av3 = jnp.where(upd, v, av3)
                ap3 = jnp.where(upd, p, ap3)
                ptr3 = ptr3 + jnp.where(active, 1, 0).astype(jnp.int32)
            done = ~jnp.any(ptr3 < e2)
            blocked = (~jnp.any((ptr3 < e2) & (ptr3 < c1))) & (~done)

            def emit_adv(args):
                gi_c, s_c, e_c, av_c, ap_c = args
                res = jnp.where(e_c > s_c, ap_c, jnp.int32(0))
                obuf[pl.ds(gi_c * 16, 16)] = res
                gi_n = gi_c + 1
                base = gi_n * 16
                s_n = plsc.load_gather(sbuf, [base + lane])
                e_n = plsc.load_gather(sbuf, [base + lane + 1])
                return (gi_n, s_n, e_n, s_n,
                        jnp.full((16,), INT_MAX, jnp.int32),
                        jnp.zeros((16,), jnp.int32))

            def stay(args):
                gi_c, s_c, e_c, av_c, ap_c = args
                return (gi_c, s_c, e_c, ptr3, av_c, ap_c)

            gi3, s3, e3, ptr4, av4, ap4 = lax.cond(
                done, emit_adv, stay, (gi2, s2, e2, av3, ap3))
            return (~blocked, gi3, s3, e3, ptr4, av4, ap4)

        go, gi2, s2, e2, ptr2, av2, ap2 = lax.while_loop(
            in_cond, in_body,
            (jnp.bool_(True), gi, s, e, ptr, accv, accp))
        return (gi2, c1, s2, e2, ptr2, av2, ap2)

    init = (jnp.int32(0), c0_init, s0, e0, s0,
            jnp.full((16,), INT_MAX, jnp.int32),
            jnp.zeros((16,), jnp.int32))
    lax.while_loop(outer_cond, outer_body, init)

    pltpu.sync_copy(obuf, out_hbm.at[pl.ds(nbase, NPW)])


def kernel(flat_node2pin_start, flat_node2pin, sorted_pin_map):
    start_pad = jnp.pad(flat_node2pin_start,
                        (0, NW * NPW + SPAD - (NN + 1)),
                        constant_values=NP)
    flat_pad = jnp.pad(flat_node2pin, (0, CH + 8))
    out = _segmin_kernel(start_pad, flat_pad, sorted_pin_map)
    return out[:NN]


# speedup vs baseline: 1392.8972x; 1.0903x over previous
import functools

import jax
import jax.numpy as jnp
from jax import lax
from jax.experimental import pallas as pl
from jax.experimental.pallas import tpu as pltpu
from jax.experimental.pallas import tpu_sc as plsc

NN = 100000
NP = 1600000
NW = 32
NPW = 3136
NGRP = NPW // 16
KU = 16
CH = 16384
CROWS = CH // 128
SPAD = 3168
INT_MAX = 2**31 - 1


@functools.partial(
    pl.kernel,
    mesh=plsc.VectorSubcoreMesh(core_axis_name="c", subcore_axis_name="s"),
    compiler_params=pltpu.CompilerParams(needs_layout_passes=False),
    out_type=jax.ShapeDtypeStruct((NW * NPW,), jnp.int32),
    scratch_types=[
        pltpu.VMEM((SPAD,), jnp.int32),
        pltpu.VMEM((CH,), jnp.int32),
        pltpu.VMEM((CH,), jnp.int32),
        pltpu.VMEM((CH,), jnp.int32),
        pltpu.VMEM((CH,), jnp.int32),
        pltpu.VMEM((NPW,), jnp.int32),
        pltpu.SemaphoreType.DMA,
        pltpu.SemaphoreType.DMA,
    ],
)
def _segmin_kernel(start_hbm, flat_hbm, spm_hbm, out_hbm,
                   sbuf, fbufA, vbufA, fbufB, vbufB, obuf, semA, semB):
    cid = lax.axis_index("c")
    sid = lax.axis_index("s")
    w = sid * 2 + cid
    nbase = w * NPW
    pltpu.sync_copy(start_hbm.at[pl.ds(nbase, SPAD)], sbuf)

    lane = lax.iota(jnp.int32, 16)
    p0 = sbuf[pl.ds(0, 16)][0]
    p1 = sbuf[pl.ds(NPW, 16)][0]
    s0 = plsc.load_gather(sbuf, [lane])
    e0 = plsc.load_gather(sbuf, [lane + 1])
    cinit = (p0 // 8) * 8

    def stage(c0, fbuf, vbuf, sem):
        pltpu.sync_copy(flat_hbm.at[pl.ds(pl.multiple_of(c0, 8), CH)], fbuf)

        def fire(r, x):
            pltpu.make_async_copy(
                spm_hbm.at[fbuf.at[pl.ds(r * 128, 128)]],
                vbuf.at[pl.ds(r * 128, 128)], sem).start()
            return x

        lax.fori_loop(0, CROWS, fire, 0)

    def drain(fbuf, vbuf, sem):
        def one(r, x):
            pltpu.make_async_copy(
                spm_hbm.at[fbuf.at[pl.ds(r * 128, 128)]],
                vbuf.at[pl.ds(r * 128, 128)], sem).wait()
            return x

        lax.fori_loop(0, CROWS, one, 0)

    def walk(c0, fbuf, vbuf, carry):
        c1 = c0 + CH

        def in_cond(st2):
            go, gi2, s2, e2, ptr2, av2, ap2 = st2
            return go & (gi2 < NGRP)

        def in_body(st2):
            go, gi2, s2, e2, ptr2, av2, ap2 = st2
            ptr3, av3, ap3 = ptr2, av2, ap2
            for _ in range(KU):
                active = (ptr3 < e2) & (ptr3 < c1)
                idx = jnp.clip(ptr3 - c0, 0, CH - 1)
                v = plsc.load_gather(vbuf, [idx], mask=active)
                p = plsc.load_gather(fbuf, [idx], mask=active)
                upd = active & (v < av3)
                av3 = jnp.where(upd, v, av3)
                ap3 = jnp.where(upd, p, ap3)
                ptr3 = ptr3 + jnp.where(active, 1, 0).astype(jnp.int32)
            done = ~jnp.any(ptr3 < e2)
            blocked = (~jnp.any((ptr3 < e2) & (ptr3 < c1))) & (~done)

            def emit_adv(args):
                gi_c, s_c, e_c, av_c, ap_c = args
                res = jnp.where(e_c > s_c, ap_c, jnp.int32(0))
                obuf[pl.ds(gi_c * 16, 16)] = res
                gi_n = gi_c + 1
                base = gi_n * 16
                s_n = plsc.load_gather(sbuf, [base + lane])
                e_n = plsc.load_gather(sbuf, [base + lane + 1])
                return (gi_n, s_n, e_n, s_n,
                        jnp.full((16,), INT_MAX, jnp.int32),
                        jnp.zeros((16,), jnp.int32))

            def stay(args):
                gi_c, s_c, e_c, av_c, ap_c = args
                return (gi_c, s_c, e_c, ptr3, av_c, ap_c)

            gi3, s3, e3, ptr4, av4, ap4 = lax.cond(
                done, emit_adv, stay, (gi2, s2, e2, av3, ap3))
            return (~blocked, gi3, s3, e3, ptr4, av4, ap4)

        st = (jnp.bool_(True),) + carry
        st = lax.while_loop(in_cond, in_body, st)
        return st[1:]

    nchunk = lax.max((p1 - cinit + CH - 1) // CH, jnp.int32(1))
    npair = (nchunk + 1) // 2

    stage(cinit, fbufA, vbufA, semA)

    def pair_body(j, carry):
        c0 = cinit + j * (2 * CH)
        cb = c0 + CH

        @pl.when(cb < p1)
        def _():
            stage(cb, fbufB, vbufB, semB)

        drain(fbufA, vbufA, semA)
        carry2 = walk(c0, fbufA, vbufA, carry)

        @pl.when(c0 + 2 * CH < p1)
        def _():
            stage(c0 + 2 * CH, fbufA, vbufA, semA)

        def odd_chunk(car):
            drain(fbufB, vbufB, semB)
            return walk(cb, fbufB, vbufB, car)

        return lax.cond(cb < p1, odd_chunk, lambda car: car, carry2)

    carry0 = (jnp.int32(0), s0, e0, s0,
              jnp.full((16,), INT_MAX, jnp.int32),
              jnp.zeros((16,), jnp.int32))
    lax.fori_loop(0, npair, pair_body, carry0)

    pltpu.sync_copy(obuf, out_hbm.at[pl.ds(nbase, NPW)])


def kernel(flat_node2pin_start, flat_node2pin, sorted_pin_map):
    start_pad = jnp.pad(flat_node2pin_start,
                        (0, NW * NPW + SPAD - (NN + 1)),
                        constant_values=NP)
    flat_pad = jnp.pad(flat_node2pin, (0, 2 * CH + 8))
    out = _segmin_kernel(start_pad, flat_pad, sorted_pin_map)
    return out[:NN]
